# baseline (speedup 1.0000x reference)
import functools
import jax
import jax.numpy as jnp
from jax import lax
from jax.experimental import pallas as pl
from jax.experimental.pallas import tpu as pltpu
from jax.experimental.pallas import tpu_sc as plsc

D = 64
DP = 128
G = 128


def _rsqrt(y):
    i = lax.bitcast_convert_type(y, jnp.int32)
    g = lax.bitcast_convert_type(jnp.int32(0x5F3759DF) - (i >> 1), jnp.float32)
    yh = 0.5 * y
    for _ in range(2):
        g = g * (1.5 - yh * g * g)
    return g


_GATHER_DNUMS = lax.GatherDimensionNumbers(
    offset_dims=(), collapsed_slice_dims=(0,), start_index_map=(0,))


def _shuffle(v, idx):
    return lax.gather(v, idx[:, None], _GATHER_DNUMS, slice_sizes=(1,),
                      mode=lax.GatherScatterMode.PROMISE_IN_BOUNDS)


def _lane_sum(v):
    lanes = lax.iota(jnp.int32, 16)
    for k in (8, 4, 2, 1):
        v = v + _shuffle(v, lanes ^ k)
    return v


def _make_kernel(n_total, n_pos):
    info = plsc.get_sparse_core_info()
    nw = info.num_cores * info.num_subcores
    rows_per_w = n_total // nw
    n_chunks = rows_per_w // G
    mesh = plsc.VectorSubcoreMesh(core_axis_name="c", subcore_axis_name="s")

    @functools.partial(
        pl.kernel,
        out_type=jax.ShapeDtypeStruct((n_total, D), jnp.float32),
        mesh=mesh,
        scratch_types=[
            [pltpu.VMEM((G,), jnp.int32)] * 4,
            [pltpu.VMEM((G,), jnp.int32)] * 4,
            [pltpu.VMEM((G, DP), jnp.float32)] * 2,
            [pltpu.VMEM((G, D), jnp.float32)] * 2,
            pltpu.VMEM((n_pos, DP), jnp.float32),
            pltpu.VMEM((D,), jnp.float32),
            pltpu.VMEM((D,), jnp.float32),
            [pltpu.SemaphoreType.DMA] * 4,
            [pltpu.SemaphoreType.DMA] * 4,
            [pltpu.SemaphoreType.DMA] * 2,
            [pltpu.SemaphoreType.DMA] * 2,
        ],
        compiler_params=pltpu.CompilerParams(use_tc_tiling_on_sc=True,
                                             needs_layout_passes=False),
    )
    def k(seq_hbm, pid_hbm, item_hbm, pos_hbm, gamma_hbm, beta_hbm, out_hbm,
          idx_i, idx_p, rows_i, out_v, pos_v, gam_v, bet_v,
          sem_xi, sem_xp, sem_g, sem_o):
        wid = lax.axis_index("s") * info.num_cores + lax.axis_index("c")
        w_base = wid * rows_per_w

        pltpu.sync_copy(gamma_hbm, gam_v)
        pltpu.sync_copy(beta_hbm, bet_v)
        pltpu.sync_copy(pos_hbm, pos_v)
        gb = [(gam_v[pl.ds(c * 16, 16)], bet_v[pl.ds(c * 16, 16)])
              for c in range(D // 16)]
        lanes = lax.iota(jnp.int32, 16)
        cvecs = [lanes + (c * 16) for c in range(D // 16)]

        def idx_start(g, gi):
            pltpu.async_copy(seq_hbm.at[pl.ds(w_base + g * G, G)],
                             idx_i[gi], sem_xi[gi])
            pltpu.async_copy(pid_hbm.at[pl.ds(w_base + g * G, G)],
                             idx_p[gi], sem_xp[gi])

        def idx_wait(g, gi):
            pltpu.make_async_copy(seq_hbm.at[pl.ds(w_base + g * G, G)],
                                  idx_i[gi], sem_xi[gi]).wait()
            pltpu.make_async_copy(pid_hbm.at[pl.ds(w_base + g * G, G)],
                                  idx_p[gi], sem_xp[gi]).wait()

        def gather_start(gi, b):
            pltpu.async_copy(item_hbm.at[idx_i[gi]], rows_i[b], sem_g[b])

        def gather_wait(gi, b):
            pltpu.make_async_copy(item_hbm.at[idx_i[gi]],
                                  rows_i[b], sem_g[b]).wait()

        def out_start(g, b):
            pltpu.async_copy(out_v[b],
                             out_hbm.at[pl.ds(w_base + g * G, G)], sem_o[b])

        def out_wait(g, b):
            pltpu.make_async_copy(out_v[b],
                                  out_hbm.at[pl.ds(w_base + g * G, G)],
                                  sem_o[b]).wait()

        def compute(gi, b):
            @plsc.parallel_loop(0, G, unroll=4)
            def row_body(r):
                p = plsc.load_gather(idx_p[gi], [jnp.broadcast_to(r, (16,))])
                x = [rows_i[b][r, pl.ds(c * 16, 16)]
                     + plsc.load_gather(pos_v, [p, cvecs[c]])
                     for c in range(D // 16)]
                s = _lane_sum((x[0] + x[1]) + (x[2] + x[3]))
                q = _lane_sum((x[0] * x[0] + x[1] * x[1])
                              + (x[2] * x[2] + x[3] * x[3]))
                mean = s * (1.0 / D)
                var = q * (1.0 / D) - mean * mean
                rstd = _rsqrt(var + 1e-5)
                for c in range(D // 16):
                    out_v[b][r, pl.ds(c * 16, 16)] = (
                        (x[c] - mean) * rstd * gb[c][0] + gb[c][1])

        def step(g, b, gi, *, head=False, next_gather=True, prefetch_idx=True):
            gather_wait(gi, b)
            if not head:
                out_wait(g - 2, b)
            if next_gather:
                idx_wait(g + 1, (gi + 1) % 4)
                gather_start((gi + 1) % 4, b ^ 1)
            compute(gi, b)
            if prefetch_idx:
                idx_start(g + 4, gi)
            out_start(g, b)

        nc = n_chunks
        for g in range(4):
            idx_start(jnp.int32(g), g)
        idx_wait(jnp.int32(0), 0)
        gather_start(0, 0)
        step(jnp.int32(0), 0, 0, head=True)
        step(jnp.int32(1), 1, 1, head=True)
        step(jnp.int32(2), 0, 2)
        step(jnp.int32(3), 1, 3)

        @pl.loop(4, nc - 4, step=4)
        def main(g0):
            step(g0, 0, 0)
            step(g0 + 1, 1, 1)
            step(g0 + 2, 0, 2)
            step(g0 + 3, 1, 3)

        step(jnp.int32(nc - 4), 0, 0, prefetch_idx=False)
        step(jnp.int32(nc - 3), 1, 1, prefetch_idx=False)
        step(jnp.int32(nc - 2), 0, 2, prefetch_idx=False)
        step(jnp.int32(nc - 1), 1, 3, prefetch_idx=False, next_gather=False)
        out_wait(jnp.int32(nc - 2), 0)
        out_wait(jnp.int32(nc - 1), 1)

    return k


def kernel(input_sequence, position_ids, item_table, pos_table, ln_gamma, ln_beta):
    b, s = input_sequence.shape
    n = b * s
    seq = input_sequence.reshape(n)
    pid = position_ids.reshape(n)
    item_pad = jnp.pad(item_table, ((0, 0), (0, DP - D)))
    pos_pad = jnp.pad(pos_table, ((0, 0), (0, DP - D)))
    k = _make_kernel(n, pos_table.shape[0])
    out = k(seq, pid, item_pad, pos_pad, ln_gamma, ln_beta)
    return out.reshape(b, s, D)


# speedup vs baseline: 1.0165x; 1.0165x over previous
import functools
import jax
import jax.numpy as jnp
from jax import lax
from jax.experimental import pallas as pl
from jax.experimental.pallas import tpu as pltpu
from jax.experimental.pallas import tpu_sc as plsc

D = 64
DP = 128
G = 128


def _rsqrt(y):
    i = lax.bitcast_convert_type(y, jnp.int32)
    g = lax.bitcast_convert_type(jnp.int32(0x5F3759DF) - (i >> 1), jnp.float32)
    yh = 0.5 * y
    for _ in range(3):
        g = g * (1.5 - yh * g * g)
    return g


_GATHER_DNUMS = lax.GatherDimensionNumbers(
    offset_dims=(), collapsed_slice_dims=(0,), start_index_map=(0,))


def _shuffle(v, idx):
    return lax.gather(v, idx[:, None], _GATHER_DNUMS, slice_sizes=(1,),
                      mode=lax.GatherScatterMode.PROMISE_IN_BOUNDS)


def _lane_sum(v):
    lanes = lax.iota(jnp.int32, 16)
    for k in (8, 4, 2, 1):
        v = v + _shuffle(v, lanes ^ k)
    return v


def _make_kernel(n_total, n_pos):
    info = plsc.get_sparse_core_info()
    nw = info.num_cores * info.num_subcores
    rows_per_w = n_total // nw
    n_chunks = rows_per_w // G
    mesh = plsc.VectorSubcoreMesh(core_axis_name="c", subcore_axis_name="s")

    @functools.partial(
        pl.kernel,
        out_type=jax.ShapeDtypeStruct((n_total, D), jnp.float32),
        mesh=mesh,
        scratch_types=[
            [pltpu.VMEM((G,), jnp.int32)] * 4,
            [pltpu.VMEM((G,), jnp.int32)] * 4,
            [pltpu.VMEM((G, DP), jnp.float32)] * 2,
            [pltpu.VMEM((G, D), jnp.float32)] * 2,
            pltpu.VMEM((n_pos, DP), jnp.float32),
            pltpu.VMEM((D,), jnp.float32),
            pltpu.VMEM((D,), jnp.float32),
            [pltpu.SemaphoreType.DMA] * 4,
            [pltpu.SemaphoreType.DMA] * 4,
            [pltpu.SemaphoreType.DMA] * 2,
            [pltpu.SemaphoreType.DMA] * 2,
        ],
        compiler_params=pltpu.CompilerParams(use_tc_tiling_on_sc=True,
                                             needs_layout_passes=False),
    )
    def k(seq_hbm, pid_hbm, item_hbm, pos_hbm, gamma_hbm, beta_hbm, out_hbm,
          idx_i, idx_p, rows_i, out_v, pos_v, gam_v, bet_v,
          sem_xi, sem_xp, sem_g, sem_o):
        wid = lax.axis_index("s") * info.num_cores + lax.axis_index("c")
        w_base = wid * rows_per_w

        pltpu.sync_copy(gamma_hbm, gam_v)
        pltpu.sync_copy(beta_hbm, bet_v)
        pltpu.sync_copy(pos_hbm, pos_v)
        gb = [(gam_v[pl.ds(c * 16, 16)], bet_v[pl.ds(c * 16, 16)])
              for c in range(D // 16)]
        lanes = lax.iota(jnp.int32, 16)
        cvecs = [lanes + (c * 16) for c in range(D // 16)]

        def idx_start(g, gi):
            pltpu.async_copy(seq_hbm.at[pl.ds(w_base + g * G, G)],
                             idx_i[gi], sem_xi[gi])
            pltpu.async_copy(pid_hbm.at[pl.ds(w_base + g * G, G)],
                             idx_p[gi], sem_xp[gi])

        def idx_wait(g, gi):
            pltpu.make_async_copy(seq_hbm.at[pl.ds(w_base + g * G, G)],
                                  idx_i[gi], sem_xi[gi]).wait()
            pltpu.make_async_copy(pid_hbm.at[pl.ds(w_base + g * G, G)],
                                  idx_p[gi], sem_xp[gi]).wait()

        def gather_start(gi, b):
            pltpu.async_copy(item_hbm.at[idx_i[gi]], rows_i[b], sem_g[b])

        def gather_wait(gi, b):
            pltpu.make_async_copy(item_hbm.at[idx_i[gi]],
                                  rows_i[b], sem_g[b]).wait()

        def out_start(g, b):
            pltpu.async_copy(out_v[b],
                             out_hbm.at[pl.ds(w_base + g * G, G)], sem_o[b])

        def out_wait(g, b):
            pltpu.make_async_copy(out_v[b],
                                  out_hbm.at[pl.ds(w_base + g * G, G)],
                                  sem_o[b]).wait()

        def compute(gi, b):
            @plsc.parallel_loop(0, G, unroll=4)
            def row_body(r):
                p = plsc.load_gather(idx_p[gi], [jnp.broadcast_to(r, (16,))])
                x = [rows_i[b][r, pl.ds(c * 16, 16)]
                     + plsc.load_gather(pos_v, [p, cvecs[c]])
                     for c in range(D // 16)]
                s = _lane_sum((x[0] + x[1]) + (x[2] + x[3]))
                q = _lane_sum((x[0] * x[0] + x[1] * x[1])
                              + (x[2] * x[2] + x[3] * x[3]))
                mean = s * (1.0 / D)
                var = q * (1.0 / D) - mean * mean
                rstd = _rsqrt(var + 1e-5)
                for c in range(D // 16):
                    out_v[b][r, pl.ds(c * 16, 16)] = (
                        (x[c] - mean) * rstd * gb[c][0] + gb[c][1])

        def step(g, b, gi, *, head=False, next_gather=True, prefetch_idx=True):
            gather_wait(gi, b)
            if not head:
                out_wait(g - 2, b)
            if next_gather:
                idx_wait(g + 1, (gi + 1) % 4)
                gather_start((gi + 1) % 4, b ^ 1)
            compute(gi, b)
            if prefetch_idx:
                idx_start(g + 4, gi)
            out_start(g, b)

        nc = n_chunks
        for g in range(4):
            idx_start(jnp.int32(g), g)
        idx_wait(jnp.int32(0), 0)
        gather_start(0, 0)
        step(jnp.int32(0), 0, 0, head=True)
        step(jnp.int32(1), 1, 1, head=True)
        step(jnp.int32(2), 0, 2)
        step(jnp.int32(3), 1, 3)

        @pl.loop(4, nc - 4, step=4)
        def main(g0):
            step(g0, 0, 0)
            step(g0 + 1, 1, 1)
            step(g0 + 2, 0, 2)
            step(g0 + 3, 1, 3)

        step(jnp.int32(nc - 4), 0, 0, prefetch_idx=False)
        step(jnp.int32(nc - 3), 1, 1, prefetch_idx=False)
        step(jnp.int32(nc - 2), 0, 2, prefetch_idx=False)
        step(jnp.int32(nc - 1), 1, 3, prefetch_idx=False, next_gather=False)
        out_wait(jnp.int32(nc - 2), 0)
        out_wait(jnp.int32(nc - 1), 1)

    return k


def kernel(input_sequence, position_ids, item_table, pos_table, ln_gamma, ln_beta):
    b, s = input_sequence.shape
    n = b * s
    seq = input_sequence.reshape(n)
    pid = position_ids.reshape(n)
    item_pad = jnp.pad(item_table, ((0, 0), (0, DP - D)))
    pos_pad = jnp.pad(pos_table, ((0, 0), (0, DP - D)))
    k = _make_kernel(n, pos_table.shape[0])
    out = k(seq, pid, item_pad, pos_pad, ln_gamma, ln_beta)
    return out.reshape(b, s, D)
